# baseline (speedup 1.0000x reference)
import functools

import jax
import jax.numpy as jnp
from jax import lax
from jax.experimental import pallas as pl
from jax.experimental.pallas import tpu as pltpu
from jax.experimental.pallas import tpu_sc as plsc

_N = 10000
_D = 128
_E = 320000
_MAXBP = 3
_EPS = 1e-5

_NC = 2
_NS = 16
_NW = _NC * _NS
_K = 80
_NSTEP = _E // _NW // _K
_WCH = 80
_NCHK = _N // _WCH

_MESH = plsc.VectorSubcoreMesh(core_axis_name="c", subcore_axis_name="s")
_PREC = lax.Precision.HIGHEST


def _dot(a, b):
    return jnp.dot(a, b, precision=_PREC, preferred_element_type=jnp.float32)


def _bn_in(h, g, b):
    n = h.shape[0]
    hc = h.reshape(8, n // 8, _D)
    mu = jnp.sum(jnp.sum(hc, axis=1), axis=0) * (1.0 / n)
    dv = h - mu
    dc = (dv * dv).reshape(8, n // 8, _D)
    var = jnp.sum(jnp.sum(dc, axis=1), axis=0) * (1.0 / n)
    return g * dv / jnp.sqrt(var + _EPS) + b


def _sc_spmm(y, src3, dst3, zrows):
    @functools.partial(
        pl.kernel,
        out_type=jax.ShapeDtypeStruct((_NC, _N, _D), jnp.float32),
        mesh=_MESH,
        scratch_types=[
            pltpu.VMEM((_NSTEP, _K), jnp.int32),
            pltpu.VMEM((_NSTEP, _K), jnp.int32),
            pltpu.VMEM((_K, _D), jnp.float32),
            pltpu.VMEM_SHARED((_N, _D), jnp.float32),
            pltpu.SemaphoreType.DMA,
        ],
    )
    def k(y_hbm, src_hbm, dst_hbm, z_hbm, out_hbm, src_v, dst_v, rows_v,
          acc, sem):
        c = lax.axis_index("c")
        s = lax.axis_index("s")
        tile = c * _NS + s

        pltpu.sync_copy(src_hbm.at[tile], src_v)
        pltpu.sync_copy(dst_hbm.at[tile], dst_v)
        pltpu.sync_copy(z_hbm, rows_v)

        @pl.loop(0, pl.cdiv(_NCHK, _NS))
        def _(z):
            ch = s + z * _NS

            @pl.when(ch < _NCHK)
            def _():
                pltpu.sync_copy(rows_v, acc.at[pl.ds(ch * _WCH, _WCH)])

        plsc.subcore_barrier()

        @pl.loop(0, _NSTEP)
        def _(j):
            pltpu.async_copy(y_hbm.at[src_v.at[j]], rows_v, sem).wait()
            pltpu.sync_copy(rows_v, acc.at[dst_v.at[j]], add=True)

        plsc.subcore_barrier()

        @pl.loop(0, pl.cdiv(_NCHK, _NS))
        def _(z):
            ch = s + z * _NS

            @pl.when(ch < _NCHK)
            def _():
                pltpu.sync_copy(acc.at[pl.ds(ch * _WCH, _WCH)], rows_v)
                pltpu.sync_copy(rows_v, out_hbm.at[c, pl.ds(ch * _WCH, _WCH)])

    return k(y, src3, dst3, zrows)


def _sc_degree(dst3, zrows, ones):
    @functools.partial(
        pl.kernel,
        out_type=jax.ShapeDtypeStruct((_NC, _N, _D), jnp.float32),
        mesh=_MESH,
        scratch_types=[
            pltpu.VMEM((_NSTEP, _K), jnp.int32),
            pltpu.VMEM((_K, _D), jnp.float32),
            pltpu.VMEM((_WCH, _D), jnp.float32),
            pltpu.VMEM_SHARED((_N, _D), jnp.float32),
        ],
    )
    def k(dst_hbm, z_hbm, one_hbm, out_hbm, dst_v, ones_v, zb_v, acc):
        c = lax.axis_index("c")
        s = lax.axis_index("s")
        tile = c * _NS + s

        pltpu.sync_copy(dst_hbm.at[tile], dst_v)
        pltpu.sync_copy(z_hbm, zb_v)
        pltpu.sync_copy(one_hbm, ones_v)

        @pl.loop(0, pl.cdiv(_NCHK, _NS))
        def _(z):
            ch = s + z * _NS

            @pl.when(ch < _NCHK)
            def _():
                pltpu.sync_copy(zb_v, acc.at[pl.ds(ch * _WCH, _WCH)])

        plsc.subcore_barrier()

        @pl.loop(0, _NSTEP)
        def _(j):
            pltpu.sync_copy(ones_v, acc.at[dst_v.at[j]], add=True)

        plsc.subcore_barrier()

        @pl.loop(0, pl.cdiv(_NCHK, _NS))
        def _(z):
            ch = s + z * _NS

            @pl.when(ch < _NCHK)
            def _():
                pltpu.sync_copy(acc.at[pl.ds(ch * _WCH, _WCH)], zb_v)
                pltpu.sync_copy(zb_v, out_hbm.at[c, pl.ds(ch * _WCH, _WCH)])

    return k(dst3, zrows, ones)


def _tc_prelude(x, W_in, b_in, g1, bb1):
    def k(x_r, w_r, b_r, g_r, bb_r, h_r):
        h = jnp.maximum(_dot(x_r[...], w_r[...].T) + b_r[...], 0.0)
        h_r[...] = _bn_in(h, g_r[...], bb_r[...])

    return pl.pallas_call(
        k, out_shape=jax.ShapeDtypeStruct((_N, _D), jnp.float32)
    )(x, W_in, b_in, g1, bb1)


def _tc_prepare(h0, W_gcn, degp):
    def k(h_r, w_r, p_r, y_r, dinv_r):
        dinv = 1.0 / jnp.sqrt(1.0 + p_r[0] + p_r[1])
        dinv_r[...] = dinv
        y_r[...] = dinv * _dot(h_r[...], w_r[...].T)

    return pl.pallas_call(
        k,
        out_shape=(
            jax.ShapeDtypeStruct((_N, _D), jnp.float32),
            jax.ShapeDtypeStruct((_N, _D), jnp.float32),
        ),
    )(h0, W_gcn, degp)


_GB = 2000


def _tc_gru(h, y, dinv, part, W_ih, b_ih, W_hh, b_hh, b_gcn):
    def k(h_r, y_r, dinv_r, p_r, wih_r, bih_r, whh_r, bhh_r, bg_r,
          hn_r, ss_r):
        i = pl.program_id(0)
        hh = h_r[...]
        cc = dinv_r[...] * (p_r[0] + p_r[1] + y_r[...]) + bg_r[...]
        wih = wih_r[...]
        whh = whh_r[...]
        bih = bih_r[...]
        bhh = bhh_r[...]
        r = jax.nn.sigmoid(_dot(hh, wih[0:_D].T) + bih[0:_D]
                           + _dot(cc, whh[0:_D].T) + bhh[0:_D])
        z = jax.nn.sigmoid(_dot(hh, wih[_D:2 * _D].T) + bih[_D:2 * _D]
                           + _dot(cc, whh[_D:2 * _D].T) + bhh[_D:2 * _D])
        n_ = jnp.tanh(_dot(hh, wih[2 * _D:].T) + bih[2 * _D:]
                      + r * (_dot(cc, whh[2 * _D:].T) + bhh[2 * _D:]))
        hn = (1.0 - z) * n_ + z * cc
        hn_r[...] = hn

        @pl.when(i == 0)
        def _():
            ss_r[...] = jnp.zeros((1, _D), jnp.float32)

        ss_r[...] += jnp.sum(hn, axis=0, keepdims=True)

    row = lambda i: (i, 0)
    whole = lambda i: (0, 0)
    return pl.pallas_call(
        k,
        grid=(_N // _GB,),
        in_specs=[
            pl.BlockSpec((_GB, _D), row),
            pl.BlockSpec((_GB, _D), row),
            pl.BlockSpec((_GB, _D), row),
            pl.BlockSpec((2, _GB, _D), lambda i: (0, i, 0)),
            pl.BlockSpec((3 * _D, _D), whole),
            pl.BlockSpec((3 * _D,), lambda i: (0,)),
            pl.BlockSpec((3 * _D, _D), whole),
            pl.BlockSpec((3 * _D,), lambda i: (0,)),
            pl.BlockSpec((_D,), lambda i: (0,)),
        ],
        out_specs=(
            pl.BlockSpec((_GB, _D), row),
            pl.BlockSpec((1, _D), whole),
        ),
        out_shape=(
            jax.ShapeDtypeStruct((_N, _D), jnp.float32),
            jax.ShapeDtypeStruct((1, _D), jnp.float32),
        ),
    )(h, y, dinv, part, W_ih, b_ih, W_hh, b_hh, b_gcn)


def _tc_var(hn, ss):
    def k(hn_r, ss_r, sq_r):
        i = pl.program_id(0)
        d = hn_r[...] - ss_r[...] * (1.0 / _N)

        @pl.when(i == 0)
        def _():
            sq_r[...] = jnp.zeros((1, _D), jnp.float32)

        sq_r[...] += jnp.sum(d * d, axis=0, keepdims=True)

    row = lambda i: (i, 0)
    whole = lambda i: (0, 0)
    return pl.pallas_call(
        k,
        grid=(_N // _GB,),
        in_specs=[
            pl.BlockSpec((_GB, _D), row),
            pl.BlockSpec((1, _D), whole),
        ],
        out_specs=pl.BlockSpec((1, _D), whole),
        out_shape=jax.ShapeDtypeStruct((1, _D), jnp.float32),
    )(hn, ss)


def _tc_post(hn, ss, sq, g2, bb2, max_x, W_gcn, dinv):
    def k(hn_r, ss_r, sq_r, g_r, bb_r, mx_r, w_r, dinv_r, hb_r, mxo_r, y_r):
        mu = ss_r[...] * (1.0 / _N)
        var = sq_r[...] * (1.0 / _N)
        hb = g_r[...] * (hn_r[...] - mu) / jnp.sqrt(var + _EPS) + bb_r[...]
        hb_r[...] = hb
        mxo_r[...] = jnp.maximum(mx_r[...], hb)
        y_r[...] = dinv_r[...] * _dot(hb, w_r[...].T)

    row = lambda i: (i, 0)
    whole = lambda i: (0, 0)
    return pl.pallas_call(
        k,
        grid=(_N // _GB,),
        in_specs=[
            pl.BlockSpec((_GB, _D), row),
            pl.BlockSpec((1, _D), whole),
            pl.BlockSpec((1, _D), whole),
            pl.BlockSpec((_D,), lambda i: (0,)),
            pl.BlockSpec((_D,), lambda i: (0,)),
            pl.BlockSpec((_GB, _D), row),
            pl.BlockSpec((_D, _D), whole),
            pl.BlockSpec((_GB, _D), row),
        ],
        out_specs=(
            pl.BlockSpec((_GB, _D), row),
            pl.BlockSpec((_GB, _D), row),
            pl.BlockSpec((_GB, _D), row),
        ),
        out_shape=(
            jax.ShapeDtypeStruct((_N, _D), jnp.float32),
            jax.ShapeDtypeStruct((_N, _D), jnp.float32),
            jax.ShapeDtypeStruct((_N, _D), jnp.float32),
        ),
    )(hn, ss, sq, g2, bb2, max_x, W_gcn, dinv)


def _tc_head(mx, g3, bb3, W_h, b_h2, g4, bb4, W_o, b_o):
    def k(mx_r, g3_r, bb3_r, wh_r, bh_r, g4_r, bb4_r, wo_r, bo_r, o_r):
        t = _bn_in(mx_r[...], g3_r[...], bb3_r[...])
        u = _bn_in(_dot(t, wh_r[...].T) + bh_r[...], g4_r[...], bb4_r[...])
        u = jnp.maximum(u, 0.0)
        o_r[...] = jnp.sum(u * wo_r[...], axis=1, keepdims=True) + bo_r[0]

    return pl.pallas_call(
        k, out_shape=jax.ShapeDtypeStruct((_N, 1), jnp.float32)
    )(mx, g3, bb3, W_h, b_h2, g4, bb4, W_o, b_o)


def kernel(x, edge_index, W_in, b_in, g1, bb1, W_gcn, b_gcn, W_ih, b_ih,
           W_hh, b_hh, g2, bb2, g3, bb3, W_h, b_h2, g4, bb4, W_o, b_o):
    src3 = edge_index[0].reshape(_NW, _NSTEP, _K)
    dst3 = edge_index[1].reshape(_NW, _NSTEP, _K)
    zrows = jnp.zeros((_WCH, _D), jnp.float32)
    ones = jnp.ones((_K, _D), jnp.float32)

    degp = _sc_degree(dst3, zrows, ones)
    h = _tc_prelude(x, W_in, b_in, g1, bb1)
    max_x = h
    y, dinv = _tc_prepare(h, W_gcn, degp)

    for _ in range(_MAXBP):
        part = _sc_spmm(y, src3, dst3, zrows)
        hn, ss = _tc_gru(h, y, dinv, part, W_ih, b_ih, W_hh, b_hh, b_gcn)
        sq = _tc_var(hn, ss)
        h, max_x, y = _tc_post(hn, ss, sq, g2, bb2, max_x, W_gcn, dinv)

    return _tc_head(max_x, g3, bb3, W_h, b_h2, g4, bb4, W_o, b_o)


# speedup vs baseline: 1.3397x; 1.3397x over previous
import functools

import jax
import jax.numpy as jnp
from jax import lax
from jax.experimental import pallas as pl
from jax.experimental.pallas import tpu as pltpu
from jax.experimental.pallas import tpu_sc as plsc

_N = 10000
_D = 128
_E = 320000
_MAXBP = 3
_EPS = 1e-5

_NC = 2
_NS = 16
_NW = _NC * _NS
_K = 80
_NSTEP = _E // _NW // _K
_NSEC = 4
_SCH = 32
_WCH = 80
_NCHK = _N // _WCH

_MESH = plsc.VectorSubcoreMesh(core_axis_name="c", subcore_axis_name="s")
_PREC = lax.Precision.HIGHEST


def _dot(a, b):
    return jnp.dot(a, b, precision=_PREC, preferred_element_type=jnp.float32)


def _bn_in(h, g, b):
    n = h.shape[0]
    hc = h.reshape(8, n // 8, _D)
    mu = jnp.sum(jnp.sum(hc, axis=1), axis=0) * (1.0 / n)
    dv = h - mu
    dc = (dv * dv).reshape(8, n // 8, _D)
    var = jnp.sum(jnp.sum(dc, axis=1), axis=0) * (1.0 / n)
    return g * dv / jnp.sqrt(var + _EPS) + b


def _sc_spmm(y, src4, dst4, zrows):
    @functools.partial(
        pl.kernel,
        out_type=jax.ShapeDtypeStruct((_NC, _N, _D), jnp.float32),
        mesh=_MESH,
        scratch_types=[
            pltpu.VMEM((_SCH, _K), jnp.int32),
            pltpu.VMEM((_SCH, _K), jnp.int32),
            pltpu.VMEM((_K, _D), jnp.float32),
            pltpu.VMEM((_K, _D), jnp.float32),
            pltpu.VMEM_SHARED((_N, _D), jnp.float32),
            pltpu.SemaphoreType.DMA,
            pltpu.SemaphoreType.DMA,
        ],
    )
    def k(y_hbm, src_hbm, dst_hbm, z_hbm, out_hbm, src_v, dst_v,
          rows0, rows1, acc, sg0, sg1):
        c = lax.axis_index("c")
        s = lax.axis_index("s")
        tile = c * _NS + s

        pltpu.sync_copy(z_hbm, rows0)

        @pl.loop(0, pl.cdiv(_NCHK, _NS))
        def _(z):
            ch = s + z * _NS

            @pl.when(ch < _NCHK)
            def _():
                pltpu.sync_copy(rows0, acc.at[pl.ds(ch * _WCH, _WCH)])

        plsc.subcore_barrier()

        def gather(e, buf, sem):
            pltpu.async_copy(y_hbm.at[src_v.at[e]], buf, sem)

        def gwait(e, buf, sem):
            pltpu.make_async_copy(y_hbm.at[src_v.at[e]], buf, sem).wait()

        def scat(e, buf):
            pltpu.sync_copy(buf, acc.at[dst_v.at[e]], add=True)

        for sec in range(_NSEC):
            nch = _SCH if sec < _NSEC - 1 else _NSTEP - (_NSEC - 1) * _SCH
            start = nch % 2
            pltpu.sync_copy(src_hbm.at[tile, sec], src_v)
            pltpu.sync_copy(dst_hbm.at[tile, sec], dst_v)
            if start:
                pltpu.async_copy(y_hbm.at[src_v.at[0]], rows0, sg0).wait()
                scat(0, rows0)
            gather(start, rows0, sg0)

            @pl.loop(0, (nch - start) // 2)
            def _(p):
                e0 = start + 2 * p
                gwait(e0, rows0, sg0)
                gather(e0 + 1, rows1, sg1)
                scat(e0, rows0)

                @pl.when(e0 + 2 < nch)
                def _():
                    gather(e0 + 2, rows0, sg0)

                gwait(e0 + 1, rows1, sg1)
                scat(e0 + 1, rows1)

        plsc.subcore_barrier()

        @pl.loop(0, pl.cdiv(_NCHK, _NS))
        def _(z):
            ch = s + z * _NS

            @pl.when(ch < _NCHK)
            def _():
                pltpu.sync_copy(acc.at[pl.ds(ch * _WCH, _WCH)], rows0)
                pltpu.sync_copy(rows0, out_hbm.at[c, pl.ds(ch * _WCH, _WCH)])

    return k(y, src4, dst4, zrows)


def _sc_degree(dst3, zrows, ones):
    @functools.partial(
        pl.kernel,
        out_type=jax.ShapeDtypeStruct((_NC, _N, _D), jnp.float32),
        mesh=_MESH,
        scratch_types=[
            pltpu.VMEM((_NSTEP, _K), jnp.int32),
            pltpu.VMEM((_K, _D), jnp.float32),
            pltpu.VMEM((_WCH, _D), jnp.float32),
            pltpu.VMEM_SHARED((_N, _D), jnp.float32),
        ],
    )
    def k(dst_hbm, z_hbm, one_hbm, out_hbm, dst_v, ones_v, zb_v, acc):
        c = lax.axis_index("c")
        s = lax.axis_index("s")
        tile = c * _NS + s

        pltpu.sync_copy(dst_hbm.at[tile], dst_v)
        pltpu.sync_copy(z_hbm, zb_v)
        pltpu.sync_copy(one_hbm, ones_v)

        @pl.loop(0, pl.cdiv(_NCHK, _NS))
        def _(z):
            ch = s + z * _NS

            @pl.when(ch < _NCHK)
            def _():
                pltpu.sync_copy(zb_v, acc.at[pl.ds(ch * _WCH, _WCH)])

        plsc.subcore_barrier()

        @pl.loop(0, _NSTEP)
        def _(j):
            pltpu.sync_copy(ones_v, acc.at[dst_v.at[j]], add=True)

        plsc.subcore_barrier()

        @pl.loop(0, pl.cdiv(_NCHK, _NS))
        def _(z):
            ch = s + z * _NS

            @pl.when(ch < _NCHK)
            def _():
                pltpu.sync_copy(acc.at[pl.ds(ch * _WCH, _WCH)], zb_v)
                pltpu.sync_copy(zb_v, out_hbm.at[c, pl.ds(ch * _WCH, _WCH)])

    return k(dst3, zrows, ones)


def _tc_prelude(x, W_in, b_in, g1, bb1):
    def k(x_r, w_r, b_r, g_r, bb_r, h_r):
        h = jnp.maximum(_dot(x_r[...], w_r[...].T) + b_r[...], 0.0)
        h_r[...] = _bn_in(h, g_r[...], bb_r[...])

    return pl.pallas_call(
        k, out_shape=jax.ShapeDtypeStruct((_N, _D), jnp.float32)
    )(x, W_in, b_in, g1, bb1)


def _tc_prepare(h0, W_gcn, degp):
    def k(h_r, w_r, p_r, y_r, dinv_r):
        dinv = 1.0 / jnp.sqrt(1.0 + p_r[0] + p_r[1])
        dinv_r[...] = dinv
        y_r[...] = dinv * _dot(h_r[...], w_r[...].T)

    return pl.pallas_call(
        k,
        out_shape=(
            jax.ShapeDtypeStruct((_N, _D), jnp.float32),
            jax.ShapeDtypeStruct((_N, _D), jnp.float32),
        ),
    )(h0, W_gcn, degp)


_GB = 2000


def _tc_gru(h, y, dinv, part, W_ih, b_ih, W_hh, b_hh, b_gcn):
    def k(h_r, y_r, dinv_r, p_r, wih_r, bih_r, whh_r, bhh_r, bg_r,
          hn_r, ss_r):
        i = pl.program_id(0)
        hh = h_r[...]
        cc = dinv_r[...] * (p_r[0] + p_r[1] + y_r[...]) + bg_r[...]
        wih = wih_r[...]
        whh = whh_r[...]
        bih = bih_r[...]
        bhh = bhh_r[...]
        r = jax.nn.sigmoid(_dot(hh, wih[0:_D].T) + bih[0:_D]
                           + _dot(cc, whh[0:_D].T) + bhh[0:_D])
        z = jax.nn.sigmoid(_dot(hh, wih[_D:2 * _D].T) + bih[_D:2 * _D]
                           + _dot(cc, whh[_D:2 * _D].T) + bhh[_D:2 * _D])
        n_ = jnp.tanh(_dot(hh, wih[2 * _D:].T) + bih[2 * _D:]
                      + r * (_dot(cc, whh[2 * _D:].T) + bhh[2 * _D:]))
        hn = (1.0 - z) * n_ + z * cc
        hn_r[...] = hn

        @pl.when(i == 0)
        def _():
            ss_r[...] = jnp.zeros((1, _D), jnp.float32)

        ss_r[...] += jnp.sum(hn, axis=0, keepdims=True)

    row = lambda i: (i, 0)
    whole = lambda i: (0, 0)
    return pl.pallas_call(
        k,
        grid=(_N // _GB,),
        in_specs=[
            pl.BlockSpec((_GB, _D), row),
            pl.BlockSpec((_GB, _D), row),
            pl.BlockSpec((_GB, _D), row),
            pl.BlockSpec((2, _GB, _D), lambda i: (0, i, 0)),
            pl.BlockSpec((3 * _D, _D), whole),
            pl.BlockSpec((3 * _D,), lambda i: (0,)),
            pl.BlockSpec((3 * _D, _D), whole),
            pl.BlockSpec((3 * _D,), lambda i: (0,)),
            pl.BlockSpec((_D,), lambda i: (0,)),
        ],
        out_specs=(
            pl.BlockSpec((_GB, _D), row),
            pl.BlockSpec((1, _D), whole),
        ),
        out_shape=(
            jax.ShapeDtypeStruct((_N, _D), jnp.float32),
            jax.ShapeDtypeStruct((1, _D), jnp.float32),
        ),
    )(h, y, dinv, part, W_ih, b_ih, W_hh, b_hh, b_gcn)


def _tc_var(hn, ss):
    def k(hn_r, ss_r, sq_r):
        i = pl.program_id(0)
        d = hn_r[...] - ss_r[...] * (1.0 / _N)

        @pl.when(i == 0)
        def _():
            sq_r[...] = jnp.zeros((1, _D), jnp.float32)

        sq_r[...] += jnp.sum(d * d, axis=0, keepdims=True)

    row = lambda i: (i, 0)
    whole = lambda i: (0, 0)
    return pl.pallas_call(
        k,
        grid=(_N // _GB,),
        in_specs=[
            pl.BlockSpec((_GB, _D), row),
            pl.BlockSpec((1, _D), whole),
        ],
        out_specs=pl.BlockSpec((1, _D), whole),
        out_shape=jax.ShapeDtypeStruct((1, _D), jnp.float32),
    )(hn, ss)


def _tc_post(hn, ss, sq, g2, bb2, max_x, W_gcn, dinv):
    def k(hn_r, ss_r, sq_r, g_r, bb_r, mx_r, w_r, dinv_r, hb_r, mxo_r, y_r):
        mu = ss_r[...] * (1.0 / _N)
        var = sq_r[...] * (1.0 / _N)
        hb = g_r[...] * (hn_r[...] - mu) / jnp.sqrt(var + _EPS) + bb_r[...]
        hb_r[...] = hb
        mxo_r[...] = jnp.maximum(mx_r[...], hb)
        y_r[...] = dinv_r[...] * _dot(hb, w_r[...].T)

    row = lambda i: (i, 0)
    whole = lambda i: (0, 0)
    return pl.pallas_call(
        k,
        grid=(_N // _GB,),
        in_specs=[
            pl.BlockSpec((_GB, _D), row),
            pl.BlockSpec((1, _D), whole),
            pl.BlockSpec((1, _D), whole),
            pl.BlockSpec((_D,), lambda i: (0,)),
            pl.BlockSpec((_D,), lambda i: (0,)),
            pl.BlockSpec((_GB, _D), row),
            pl.BlockSpec((_D, _D), whole),
            pl.BlockSpec((_GB, _D), row),
        ],
        out_specs=(
            pl.BlockSpec((_GB, _D), row),
            pl.BlockSpec((_GB, _D), row),
            pl.BlockSpec((_GB, _D), row),
        ),
        out_shape=(
            jax.ShapeDtypeStruct((_N, _D), jnp.float32),
            jax.ShapeDtypeStruct((_N, _D), jnp.float32),
            jax.ShapeDtypeStruct((_N, _D), jnp.float32),
        ),
    )(hn, ss, sq, g2, bb2, max_x, W_gcn, dinv)


def _tc_head(mx, g3, bb3, W_h, b_h2, g4, bb4, W_o, b_o):
    def k(mx_r, g3_r, bb3_r, wh_r, bh_r, g4_r, bb4_r, wo_r, bo_r, o_r):
        t = _bn_in(mx_r[...], g3_r[...], bb3_r[...])
        u = _bn_in(_dot(t, wh_r[...].T) + bh_r[...], g4_r[...], bb4_r[...])
        u = jnp.maximum(u, 0.0)
        o_r[...] = jnp.sum(u * wo_r[...], axis=1, keepdims=True) + bo_r[0]

    return pl.pallas_call(
        k, out_shape=jax.ShapeDtypeStruct((_N, 1), jnp.float32)
    )(mx, g3, bb3, W_h, b_h2, g4, bb4, W_o, b_o)


def kernel(x, edge_index, W_in, b_in, g1, bb1, W_gcn, b_gcn, W_ih, b_ih,
           W_hh, b_hh, g2, bb2, g3, bb3, W_h, b_h2, g4, bb4, W_o, b_o):
    src3 = edge_index[0].reshape(_NW, _NSTEP, _K)
    dst3 = edge_index[1].reshape(_NW, _NSTEP, _K)
    pad = ((0, 0), (0, _NSEC * _SCH - _NSTEP), (0, 0))
    src4 = jnp.pad(src3, pad).reshape(_NW, _NSEC, _SCH, _K)
    dst4 = jnp.pad(dst3, pad).reshape(_NW, _NSEC, _SCH, _K)
    zrows = jnp.zeros((_WCH, _D), jnp.float32)
    ones = jnp.ones((_K, _D), jnp.float32)

    degp = _sc_degree(dst3, zrows, ones)
    h = _tc_prelude(x, W_in, b_in, g1, bb1)
    max_x = h
    y, dinv = _tc_prepare(h, W_gcn, degp)

    for _ in range(_MAXBP):
        part = _sc_spmm(y, src4, dst4, zrows)
        hn, ss = _tc_gru(h, y, dinv, part, W_ih, b_ih, W_hh, b_hh, b_gcn)
        sq = _tc_var(hn, ss)
        h, max_x, y = _tc_post(hn, ss, sq, g2, bb2, max_x, W_gcn, dinv)

    return _tc_head(max_x, g3, bb3, W_h, b_h2, g4, bb4, W_o, b_o)


# speedup vs baseline: 1.3423x; 1.0019x over previous
import functools

import jax
import jax.numpy as jnp
from jax import lax
from jax.experimental import pallas as pl
from jax.experimental.pallas import tpu as pltpu
from jax.experimental.pallas import tpu_sc as plsc

_N = 10000
_D = 128
_E = 320000
_MAXBP = 3
_EPS = 1e-5

_NC = 2
_NS = 16
_NW = _NC * _NS
_K = 80
_NSTEP = _E // _NW // _K
_NSEC = 4
_SCH = 32
_WCH = 80
_NCHK = _N // _WCH

_MESH = plsc.VectorSubcoreMesh(core_axis_name="c", subcore_axis_name="s")
_PREC = lax.Precision.HIGHEST


def _dot(a, b):
    return jnp.dot(a, b, precision=_PREC, preferred_element_type=jnp.float32)


def _bn_in(h, g, b):
    n = h.shape[0]
    hc = h.reshape(8, n // 8, _D)
    mu = jnp.sum(jnp.sum(hc, axis=1), axis=0) * (1.0 / n)
    dv = h - mu
    dc = (dv * dv).reshape(8, n // 8, _D)
    var = jnp.sum(jnp.sum(dc, axis=1), axis=0) * (1.0 / n)
    return g * dv / jnp.sqrt(var + _EPS) + b


def _sc_spmm(y, src4, dst4, zrows):
    @functools.partial(
        pl.kernel,
        out_type=jax.ShapeDtypeStruct((_NC, _N, _D), jnp.float32),
        mesh=_MESH,
        scratch_types=[
            pltpu.VMEM((_SCH, _K), jnp.int32),
            pltpu.VMEM((_SCH, _K), jnp.int32),
            pltpu.VMEM((_K, _D), jnp.float32),
            pltpu.VMEM((_K, _D), jnp.float32),
            pltpu.VMEM_SHARED((_N, _D), jnp.float32),
            pltpu.SemaphoreType.DMA,
            pltpu.SemaphoreType.DMA,
        ],
    )
    def k(y_hbm, src_hbm, dst_hbm, z_hbm, out_hbm, src_v, dst_v,
          rows0, rows1, acc, sg0, sg1):
        c = lax.axis_index("c")
        s = lax.axis_index("s")
        tile = c * _NS + s

        pltpu.sync_copy(z_hbm, rows0)

        @pl.loop(0, pl.cdiv(_NCHK, _NS))
        def _(z):
            ch = s + z * _NS

            @pl.when(ch < _NCHK)
            def _():
                pltpu.sync_copy(rows0, acc.at[pl.ds(ch * _WCH, _WCH)])

        plsc.subcore_barrier()

        def gather(e, buf, sem):
            pltpu.async_copy(y_hbm.at[src_v.at[e]], buf, sem)

        def gwait(e, buf, sem):
            pltpu.make_async_copy(y_hbm.at[src_v.at[e]], buf, sem).wait()

        def scat(e, buf):
            pltpu.sync_copy(buf, acc.at[dst_v.at[e]], add=True)

        for sec in range(_NSEC):
            nch = _SCH if sec < _NSEC - 1 else _NSTEP - (_NSEC - 1) * _SCH
            start = nch % 2
            pltpu.sync_copy(src_hbm.at[tile, sec], src_v)
            pltpu.sync_copy(dst_hbm.at[tile, sec], dst_v)
            if start:
                pltpu.async_copy(y_hbm.at[src_v.at[0]], rows0, sg0).wait()
                scat(0, rows0)
            gather(start, rows0, sg0)

            @pl.loop(0, (nch - start) // 2)
            def _(p):
                e0 = start + 2 * p
                gwait(e0, rows0, sg0)
                gather(e0 + 1, rows1, sg1)
                scat(e0, rows0)

                @pl.when(e0 + 2 < nch)
                def _():
                    gather(e0 + 2, rows0, sg0)

                gwait(e0 + 1, rows1, sg1)
                scat(e0 + 1, rows1)

        plsc.subcore_barrier()

        @pl.loop(0, pl.cdiv(_NCHK, _NS))
        def _(z):
            ch = s + z * _NS

            @pl.when(ch < _NCHK)
            def _():
                pltpu.sync_copy(acc.at[pl.ds(ch * _WCH, _WCH)], rows0)
                pltpu.sync_copy(rows0, out_hbm.at[c, pl.ds(ch * _WCH, _WCH)])

    return k(y, src4, dst4, zrows)


def _sc_degree(dst3, zrows, ones):
    @functools.partial(
        pl.kernel,
        out_type=jax.ShapeDtypeStruct((_NC, _N, _D), jnp.float32),
        mesh=_MESH,
        scratch_types=[
            pltpu.VMEM((_NSTEP, _K), jnp.int32),
            pltpu.VMEM((_K, _D), jnp.float32),
            pltpu.VMEM((_WCH, _D), jnp.float32),
            pltpu.VMEM_SHARED((_N, _D), jnp.float32),
            pltpu.SemaphoreType.DMA,
        ],
    )
    def k(dst_hbm, z_hbm, one_hbm, out_hbm, dst_v, ones_v, zb_v, acc, sem):
        c = lax.axis_index("c")
        s = lax.axis_index("s")
        tile = c * _NS + s

        pltpu.sync_copy(dst_hbm.at[tile], dst_v)
        pltpu.sync_copy(z_hbm, zb_v)
        pltpu.sync_copy(one_hbm, ones_v)

        @pl.loop(0, pl.cdiv(_NCHK, _NS))
        def _(z):
            ch = s + z * _NS

            @pl.when(ch < _NCHK)
            def _():
                pltpu.sync_copy(zb_v, acc.at[pl.ds(ch * _WCH, _WCH)])

        plsc.subcore_barrier()

        @pl.loop(0, _NSTEP // 4)
        def _(g):
            for i in range(4):
                pltpu.async_copy(ones_v, acc.at[dst_v.at[4 * g + i]], sem,
                                 add=True)
            for i in range(4):
                pltpu.make_async_copy(ones_v, acc.at[dst_v.at[4 * g + i]],
                                      sem).wait()

        @pl.loop(4 * (_NSTEP // 4), _NSTEP)
        def _(j):
            pltpu.sync_copy(ones_v, acc.at[dst_v.at[j]], add=True)

        plsc.subcore_barrier()

        @pl.loop(0, pl.cdiv(_NCHK, _NS))
        def _(z):
            ch = s + z * _NS

            @pl.when(ch < _NCHK)
            def _():
                pltpu.sync_copy(acc.at[pl.ds(ch * _WCH, _WCH)], zb_v)
                pltpu.sync_copy(zb_v, out_hbm.at[c, pl.ds(ch * _WCH, _WCH)])

    return k(dst3, zrows, ones)


def _tc_prelude(x, W_in, b_in, g1, bb1):
    def k(x_r, w_r, b_r, g_r, bb_r, h_r):
        h = jnp.maximum(_dot(x_r[...], w_r[...].T) + b_r[...], 0.0)
        h_r[...] = _bn_in(h, g_r[...], bb_r[...])

    return pl.pallas_call(
        k, out_shape=jax.ShapeDtypeStruct((_N, _D), jnp.float32)
    )(x, W_in, b_in, g1, bb1)


def _tc_prepare(h0, W_gcn, degp):
    def k(h_r, w_r, p_r, y_r, dinv_r):
        dinv = 1.0 / jnp.sqrt(1.0 + p_r[0] + p_r[1])
        dinv_r[...] = dinv
        y_r[...] = dinv * _dot(h_r[...], w_r[...].T)

    return pl.pallas_call(
        k,
        out_shape=(
            jax.ShapeDtypeStruct((_N, _D), jnp.float32),
            jax.ShapeDtypeStruct((_N, _D), jnp.float32),
        ),
    )(h0, W_gcn, degp)


_GB = 2000


def _tc_gru(h, y, dinv, part, W_ih, b_ih, W_hh, b_hh, b_gcn):
    def k(h_r, y_r, dinv_r, p_r, wih_r, bih_r, whh_r, bhh_r, bg_r,
          hn_r, ss_r):
        i = pl.program_id(0)
        hh = h_r[...]
        cc = dinv_r[...] * (p_r[0] + p_r[1] + y_r[...]) + bg_r[...]
        wih = wih_r[...]
        whh = whh_r[...]
        bih = bih_r[...]
        bhh = bhh_r[...]
        r = jax.nn.sigmoid(_dot(hh, wih[0:_D].T) + bih[0:_D]
                           + _dot(cc, whh[0:_D].T) + bhh[0:_D])
        z = jax.nn.sigmoid(_dot(hh, wih[_D:2 * _D].T) + bih[_D:2 * _D]
                           + _dot(cc, whh[_D:2 * _D].T) + bhh[_D:2 * _D])
        n_ = jnp.tanh(_dot(hh, wih[2 * _D:].T) + bih[2 * _D:]
                      + r * (_dot(cc, whh[2 * _D:].T) + bhh[2 * _D:]))
        hn = (1.0 - z) * n_ + z * cc
        hn_r[...] = hn

        @pl.when(i == 0)
        def _():
            ss_r[...] = jnp.zeros((1, _D), jnp.float32)

        ss_r[...] += jnp.sum(hn, axis=0, keepdims=True)

    row = lambda i: (i, 0)
    whole = lambda i: (0, 0)
    return pl.pallas_call(
        k,
        grid=(_N // _GB,),
        in_specs=[
            pl.BlockSpec((_GB, _D), row),
            pl.BlockSpec((_GB, _D), row),
            pl.BlockSpec((_GB, _D), row),
            pl.BlockSpec((2, _GB, _D), lambda i: (0, i, 0)),
            pl.BlockSpec((3 * _D, _D), whole),
            pl.BlockSpec((3 * _D,), lambda i: (0,)),
            pl.BlockSpec((3 * _D, _D), whole),
            pl.BlockSpec((3 * _D,), lambda i: (0,)),
            pl.BlockSpec((_D,), lambda i: (0,)),
        ],
        out_specs=(
            pl.BlockSpec((_GB, _D), row),
            pl.BlockSpec((1, _D), whole),
        ),
        out_shape=(
            jax.ShapeDtypeStruct((_N, _D), jnp.float32),
            jax.ShapeDtypeStruct((1, _D), jnp.float32),
        ),
    )(h, y, dinv, part, W_ih, b_ih, W_hh, b_hh, b_gcn)


def _tc_var(hn, ss):
    def k(hn_r, ss_r, sq_r):
        i = pl.program_id(0)
        d = hn_r[...] - ss_r[...] * (1.0 / _N)

        @pl.when(i == 0)
        def _():
            sq_r[...] = jnp.zeros((1, _D), jnp.float32)

        sq_r[...] += jnp.sum(d * d, axis=0, keepdims=True)

    row = lambda i: (i, 0)
    whole = lambda i: (0, 0)
    return pl.pallas_call(
        k,
        grid=(_N // _GB,),
        in_specs=[
            pl.BlockSpec((_GB, _D), row),
            pl.BlockSpec((1, _D), whole),
        ],
        out_specs=pl.BlockSpec((1, _D), whole),
        out_shape=jax.ShapeDtypeStruct((1, _D), jnp.float32),
    )(hn, ss)


def _tc_post(hn, ss, sq, g2, bb2, max_x, W_gcn, dinv):
    def k(hn_r, ss_r, sq_r, g_r, bb_r, mx_r, w_r, dinv_r, hb_r, mxo_r, y_r):
        mu = ss_r[...] * (1.0 / _N)
        var = sq_r[...] * (1.0 / _N)
        hb = g_r[...] * (hn_r[...] - mu) / jnp.sqrt(var + _EPS) + bb_r[...]
        hb_r[...] = hb
        mxo_r[...] = jnp.maximum(mx_r[...], hb)
        y_r[...] = dinv_r[...] * _dot(hb, w_r[...].T)

    row = lambda i: (i, 0)
    whole = lambda i: (0, 0)
    return pl.pallas_call(
        k,
        grid=(_N // _GB,),
        in_specs=[
            pl.BlockSpec((_GB, _D), row),
            pl.BlockSpec((1, _D), whole),
            pl.BlockSpec((1, _D), whole),
            pl.BlockSpec((_D,), lambda i: (0,)),
            pl.BlockSpec((_D,), lambda i: (0,)),
            pl.BlockSpec((_GB, _D), row),
            pl.BlockSpec((_D, _D), whole),
            pl.BlockSpec((_GB, _D), row),
        ],
        out_specs=(
            pl.BlockSpec((_GB, _D), row),
            pl.BlockSpec((_GB, _D), row),
            pl.BlockSpec((_GB, _D), row),
        ),
        out_shape=(
            jax.ShapeDtypeStruct((_N, _D), jnp.float32),
            jax.ShapeDtypeStruct((_N, _D), jnp.float32),
            jax.ShapeDtypeStruct((_N, _D), jnp.float32),
        ),
    )(hn, ss, sq, g2, bb2, max_x, W_gcn, dinv)


def _tc_head(mx, g3, bb3, W_h, b_h2, g4, bb4, W_o, b_o):
    def k(mx_r, g3_r, bb3_r, wh_r, bh_r, g4_r, bb4_r, wo_r, bo_r, o_r):
        t = _bn_in(mx_r[...], g3_r[...], bb3_r[...])
        u = _bn_in(_dot(t, wh_r[...].T) + bh_r[...], g4_r[...], bb4_r[...])
        u = jnp.maximum(u, 0.0)
        o_r[...] = jnp.sum(u * wo_r[...], axis=1, keepdims=True) + bo_r[0]

    return pl.pallas_call(
        k, out_shape=jax.ShapeDtypeStruct((_N, 1), jnp.float32)
    )(mx, g3, bb3, W_h, b_h2, g4, bb4, W_o, b_o)


def kernel(x, edge_index, W_in, b_in, g1, bb1, W_gcn, b_gcn, W_ih, b_ih,
           W_hh, b_hh, g2, bb2, g3, bb3, W_h, b_h2, g4, bb4, W_o, b_o):
    src3 = edge_index[0].reshape(_NW, _NSTEP, _K)
    dst3 = edge_index[1].reshape(_NW, _NSTEP, _K)
    pad = ((0, 0), (0, _NSEC * _SCH - _NSTEP), (0, 0))
    src4 = jnp.pad(src3, pad).reshape(_NW, _NSEC, _SCH, _K)
    dst4 = jnp.pad(dst3, pad).reshape(_NW, _NSEC, _SCH, _K)
    zrows = jnp.zeros((_WCH, _D), jnp.float32)
    ones = jnp.ones((_K, _D), jnp.float32)

    degp = _sc_degree(dst3, zrows, ones)
    h = _tc_prelude(x, W_in, b_in, g1, bb1)
    max_x = h
    y, dinv = _tc_prepare(h, W_gcn, degp)

    for _ in range(_MAXBP):
        part = _sc_spmm(y, src4, dst4, zrows)
        hn, ss = _tc_gru(h, y, dinv, part, W_ih, b_ih, W_hh, b_hh, b_gcn)
        sq = _tc_var(hn, ss)
        h, max_x, y = _tc_post(hn, ss, sq, g2, bb2, max_x, W_gcn, dinv)

    return _tc_head(max_x, g3, bb3, W_h, b_h2, g4, bb4, W_o, b_o)
